# Initial kernel scaffold; baseline (speedup 1.0000x reference)
import jax
import jax.numpy as jnp
from jax.experimental import pallas as pl


def kernel(s_feats, s_points, neighbor_indices, ln_w0, ln_w1, W_in, W_out, w_l0, w_m0, w_a, w_b):
    raise NotImplementedError("write your pallas kernel here")


# speedup vs baseline: 4.5591x; 4.5591x over previous
import functools

import jax
import jax.numpy as jnp
from jax import lax
from jax.experimental import pallas as pl
from jax.experimental.pallas import tpu as pltpu
from jax.experimental.pallas import tpu_sc as plsc

H = 8
CMID = 8
ROW = 48
NB1 = 1000
NB3 = 400


def _k1_body(x_ref, p_ref, w0_ref, w1_ref, wn_ref, ws_ref, t_ref, s_ref):
    x = x_ref[...]
    p4 = p_ref[...]
    sq = x * x
    sn = jnp.sqrt(jnp.mean(sq[:, 0:1, :], axis=(1, 2), keepdims=True) + 1e-5)
    vn = jnp.sqrt(jnp.mean(sq[:, 1:4, :], axis=(1, 2), keepdims=True) + 1e-5)
    w0 = w0_ref[...][None]
    w1 = w1_ref[...][None]
    xs = x[:, 0:1, :] / sn * w0
    xv = x[:, 1:4, :] / vn * w1
    xn = jnp.concatenate([xs, xv], axis=1)
    xf = xn.reshape(NB1 * 4, 128)
    a = (xf @ wn_ref[...]).reshape(NB1, 4, CMID)
    b = (xf @ ws_ref[...]).reshape(NB1, 4, CMID)
    pb = jnp.broadcast_to(p4[:, None, :], (NB1, 4, 4))
    t_ref[:, :, 0:CMID] = a
    t_ref[:, :, CMID:12] = pb
    s_ref[:, :, 0:CMID] = b
    s_ref[:, :, CMID:12] = pb


def _k2_body(table_hbm, idx_hbm, out_hbm, idx_v, rows_v, sem0, sem1,
             *, epw, chunk):
    c = lax.axis_index("c")
    s = lax.axis_index("s")
    wid = s * 2 + c
    base = wid * epw
    nchunks = epw // chunk
    sems = (sem0, sem1)
    pltpu.sync_copy(idx_hbm.at[pl.ds(wid * nchunks, nchunks)], idx_v)
    copies = [None, None]
    for k in range(nchunks):
        b = k % 2
        copies[b] = pltpu.async_copy(table_hbm.at[idx_v.at[k]],
                                     rows_v.at[b], sems[b])
        if k > 0:
            copies[1 - b].wait()
            pltpu.sync_copy(rows_v.at[1 - b],
                            out_hbm.at[pl.ds(base + (k - 1) * chunk, chunk)])
    copies[(nchunks - 1) % 2].wait()
    pltpu.sync_copy(rows_v.at[(nchunks - 1) % 2],
                    out_hbm.at[pl.ds(base + (nchunks - 1) * chunk, chunk)])


def _k3_body(g_ref, s_ref, wl0_ref, wm0_ref, wa_ref, wb_ref, wout_ref, o_ref):
    g = g_ref[...]
    sf = s_ref[...]
    wl0 = wl0_ref[...]
    wa3 = wa_ref[...][None]
    wb3 = wb_ref[...][None]
    wm3 = wm0_ref[...][None]

    m_s = g[:, :, 0:8] + sf[:, None, 0:8]
    m_x = g[:, :, 12:20] + sf[:, None, 12:20]
    m_y = g[:, :, 24:32] + sf[:, None, 24:32]
    m_z = g[:, :, 36:44] + sf[:, None, 36:44]

    dx = g[:, :, 8:9] - sf[:, None, 8:9]
    dy = g[:, :, 9:10] - sf[:, None, 9:10]
    dz = g[:, :, 10:11] - sf[:, None, 10:11]
    length = jnp.sqrt(dx * dx + dy * dy + dz * dz)
    inv = 1.0 / (length + 1e-8)
    ux = dx * inv
    uy = dy * inv
    uz = dz * inv

    f = 1.0 / (1.0 + uz + 1e-8)
    r00 = 1.0 - f * ux * ux
    r01 = -f * ux * uy
    r02 = -ux
    r11 = 1.0 - f * uy * uy
    r22 = 1.0 - f * (ux * ux + uy * uy)

    vrx = r00 * m_x + r01 * m_y + r02 * m_z
    vry = r01 * m_x + r11 * m_y - uy * m_z
    vrz = ux * m_x + uy * m_y + r22 * m_z
    x2 = wa3 * vrx - wb3 * vry
    y2 = wb3 * vrx + wa3 * vry
    z2 = wm3 * vrz
    vbx = r00 * x2 + r01 * y2 + ux * z2
    vby = r01 * x2 + r11 * y2 + uy * z2
    vbz = r02 * x2 - uy * y2 + r22 * z2

    s0 = jnp.sum(m_s, axis=1) * wl0
    sx = jnp.sum(vbx, axis=1)
    sy = jnp.sum(vby, axis=1)
    sz = jnp.sum(vbz, axis=1)

    wout = wout_ref[...]
    o_ref[:, 0, :] = s0 @ wout
    o_ref[:, 1, :] = sx @ wout
    o_ref[:, 2, :] = sy @ wout
    o_ref[:, 3, :] = sz @ wout


def kernel(s_feats, s_points, neighbor_indices, ln_w0, ln_w1, W_in, W_out,
           w_l0, w_m0, w_a, w_b):
    N, L, C = s_feats.shape
    f32 = jnp.float32

    p4 = jnp.concatenate([s_points, jnp.zeros((N, 1), f32)], axis=1)
    grid1 = N // NB1
    t_tab, s_tab = pl.pallas_call(
        _k1_body,
        grid=(grid1,),
        in_specs=[
            pl.BlockSpec((NB1, 4, C), lambda i: (i, 0, 0)),
            pl.BlockSpec((NB1, 4), lambda i: (i, 0)),
            pl.BlockSpec((1, C), lambda i: (0, 0)),
            pl.BlockSpec((1, C), lambda i: (0, 0)),
            pl.BlockSpec((C, CMID), lambda i: (0, 0)),
            pl.BlockSpec((C, CMID), lambda i: (0, 0)),
        ],
        out_specs=[
            pl.BlockSpec((NB1, 4, 12), lambda i: (i, 0, 0)),
            pl.BlockSpec((NB1, 4, 12), lambda i: (i, 0, 0)),
        ],
        out_shape=[
            jax.ShapeDtypeStruct((N, 4, 12), f32),
            jax.ShapeDtypeStruct((N, 4, 12), f32),
        ],
    )(s_feats, p4, ln_w0.reshape(1, C), ln_w1.reshape(1, C),
      W_in[:C, :], W_in[C:, :])

    table = t_tab.reshape(N, ROW)
    selfr = s_tab.reshape(N, ROW)

    ne = N * H
    nw = 32
    chunk = 128
    ne_pad = ((ne + nw * chunk - 1) // (nw * chunk)) * (nw * chunk)
    epw = ne_pad // nw
    nchunks = epw // chunk
    ni = neighbor_indices[:, 1:1 + H].astype(jnp.int32).reshape(ne)
    ni = jnp.concatenate([ni, jnp.zeros((ne_pad - ne,), jnp.int32)])
    ni = ni.reshape(ne_pad // chunk, chunk)

    mesh = plsc.VectorSubcoreMesh(core_axis_name="c", subcore_axis_name="s")
    gathered = pl.kernel(
        functools.partial(_k2_body, epw=epw, chunk=chunk),
        out_type=jax.ShapeDtypeStruct((ne_pad, ROW), f32),
        mesh=mesh,
        scratch_types=[
            pltpu.VMEM((nchunks, chunk), jnp.int32),
            pltpu.VMEM((2, chunk, ROW), f32),
            pltpu.SemaphoreType.DMA,
            pltpu.SemaphoreType.DMA,
        ],
        compiler_params=pltpu.CompilerParams(use_tc_tiling_on_sc=False),
    )(table, ni)

    g3 = gathered.reshape(ne_pad // H, H, ROW)
    grid3 = N // NB3
    out = pl.pallas_call(
        _k3_body,
        grid=(grid3,),
        in_specs=[
            pl.BlockSpec((NB3, H, ROW), lambda i: (i, 0, 0)),
            pl.BlockSpec((NB3, ROW), lambda i: (i, 0)),
            pl.BlockSpec((1, CMID), lambda i: (0, 0)),
            pl.BlockSpec((1, CMID), lambda i: (0, 0)),
            pl.BlockSpec((1, CMID), lambda i: (0, 0)),
            pl.BlockSpec((1, CMID), lambda i: (0, 0)),
            pl.BlockSpec((CMID, 128), lambda i: (0, 0)),
        ],
        out_specs=pl.BlockSpec((NB3, 4, 128), lambda i: (i, 0, 0)),
        out_shape=jax.ShapeDtypeStruct((N, 4, 128), f32),
    )(g3, selfr, w_l0.reshape(1, CMID), w_m0.reshape(1, CMID),
      w_a.reshape(1, CMID), w_b.reshape(1, CMID), W_out)
    return out
